# baseline (speedup 1.0000x reference)
import jax
import jax.numpy as jnp
from jax.experimental import pallas as pl
from jax.experimental.pallas import tpu as pltpu

_B, _L, _D = 8, 576, 64
_K = 1024
_N = _B * _L
_TOK = 512
_NT = _N // _TOK
_DECAY = 0.99


def _vq_body(x_ref, cb_ref, ema_ref, counts_ref,
             q_ref, loss_ref, idx_ref, cnew_ref, emanew_ref, cbnew_ref,
             dw_ref, cacc_ref):
    cb = cb_ref[...]
    c2 = jnp.sum(cb * cb, axis=1)[None, :]
    dw_ref[...] = jnp.zeros_like(dw_ref)
    cacc_ref[...] = jnp.zeros_like(cacc_ref)
    ones_col = jnp.ones((_TOK, 1), jnp.float32)

    def pass1(t, carry):
        xt = x_ref[pl.ds(t * _TOK, _TOK), :]
        xc = jax.lax.dot_general(xt, cb, (((1,), (1,)), ((), ())),
                                 preferred_element_type=jnp.float32)
        x2 = jnp.sum(xt * xt, axis=1, keepdims=True)
        d2 = x2 - 2.0 * xc + c2
        iota = jax.lax.broadcasted_iota(jnp.int32, (_TOK, _K), 1)
        m = jnp.min(d2, axis=1, keepdims=True)
        idx = jnp.min(jnp.where(d2 == m, iota, _K), axis=1, keepdims=True)
        idx_ref[pl.ds(t * _TOK, _TOK), :] = idx
        e = (idx == iota).astype(jnp.float32)
        dw_ref[...] += jax.lax.dot_general(e, xt, (((0,), (0,)), ((), ())),
                                           preferred_element_type=jnp.float32)
        cacc_ref[...] += jax.lax.dot_general(e, ones_col, (((0,), (0,)), ((), ())),
                                             preferred_element_type=jnp.float32)
        return carry

    jax.lax.fori_loop(0, _NT, pass1, 0)

    counts_new = _DECAY * counts_ref[...] + (1.0 - _DECAY) * cacc_ref[...]
    cnew_ref[...] = counts_new
    ema_new = _DECAY * ema_ref[...] + (1.0 - _DECAY) * dw_ref[...]
    emanew_ref[...] = ema_new
    cbnew = ema_new / counts_new
    cbnew_ref[...] = cbnew

    def pass2(t, loss):
        xt = x_ref[pl.ds(t * _TOK, _TOK), :]
        idx = idx_ref[pl.ds(t * _TOK, _TOK), :]
        iota = jax.lax.broadcasted_iota(jnp.int32, (_TOK, _K), 1)
        e = (idx == iota).astype(jnp.float32)
        q = jax.lax.dot_general(e, cbnew, (((1,), (0,)), ((), ())),
                                preferred_element_type=jnp.float32)
        q_ref[pl.ds(t * _TOK, _TOK), :] = q
        r = xt - q
        return loss + jnp.sum(r * r)

    loss = jax.lax.fori_loop(0, _NT, pass2, jnp.float32(0.0))
    loss_ref[...] = jnp.full((1, 1), 0.5 * loss / (_N * _D), jnp.float32)


@jax.jit
def kernel(x, codebook, ema_weight, counts):
    xf = x.reshape(_N, _D)
    counts_col = counts.reshape(_K, 1)
    q, loss, idx, cnew, emanew, cbnew = pl.pallas_call(
        _vq_body,
        out_shape=[
            jax.ShapeDtypeStruct((_N, _D), jnp.float32),
            jax.ShapeDtypeStruct((1, 1), jnp.float32),
            jax.ShapeDtypeStruct((_N, 1), jnp.int32),
            jax.ShapeDtypeStruct((_K, 1), jnp.float32),
            jax.ShapeDtypeStruct((_K, _D), jnp.float32),
            jax.ShapeDtypeStruct((_K, _D), jnp.float32),
        ],
        scratch_shapes=[
            pltpu.VMEM((_K, _D), jnp.float32),
            pltpu.VMEM((_K, 1), jnp.float32),
        ],
    )(xf, codebook, ema_weight, counts_col)
    return (q.reshape(_B, _L, _D), loss[0, 0], idx.reshape(_B, _L),
            cnew.reshape(_K), emanew, cbnew)
